# baseline (speedup 1.0000x reference)
import dataclasses

import jax
import jax.numpy as jnp
from jax import lax
from jax.experimental import pallas as pl
from jax.experimental.pallas import tpu as pltpu
from jax.experimental.pallas import tpu_sc as plsc

_N = 10000
_E = 320000
_D = 128
_NPAD = 10240
_NC = 2
_NS = 16
_NW = _NC * _NS
_CH = _E // _NW
_SL = _NPAD // _NS
_VPAD = 256
_EB = 512
_BO = 80
_CB = 2000
_CBK = 25
_F32 = jnp.float32
_I32 = jnp.int32
_HI = lax.Precision.HIGHEST


def _dot(a, b):
    return jnp.dot(a, b, preferred_element_type=_F32, precision=_HI)


def _tc_encode_body(x_ref, wne_ref, bne_ref, embp_ref, ae_ref, wl0_ref,
                    bl0_ref, asd0_ref, xe_ref, te_ref, h0_ref, ssd0_ref):
    xe = _dot(x_ref[...], wne_ref[...]) + bne_ref[...]
    xe_ref[...] = xe
    te_ref[...] = lax.dot_general(ae_ref[...], embp_ref[...],
                                  (((1,), (1,)), ((), ())),
                                  precision=_HI, preferred_element_type=_F32)
    h0 = _dot(xe, wl0_ref[...]) + bl0_ref[...]
    h0_ref[...] = h0
    ssd0_ref[...] = _dot(h0, asd0_ref[...])


def _tc_gather_body(attr_ref, embp_ref, e_ref):
    av = attr_ref[...]
    iota = lax.broadcasted_iota(_I32, (_EB, _VPAD), 1)
    oh = jnp.where(av[:, None] == iota, 1.0, 0.0).astype(_F32)
    e_ref[...] = _dot(oh, embp_ref[...])


def _bn_relu_res(x1p_ref, z_ref, prev_ref, g_ref, b_ref):
    zt = z_ref[0, :_N] + z_ref[1, :_N] + 1e-16
    s = (x1p_ref[0, :_N, :] + x1p_ref[1, :_N, :]) / zt[:, None]
    mu = jnp.mean(s, axis=0, keepdims=True)
    var = jnp.mean((s - mu) ** 2, axis=0, keepdims=True)
    xb = (s - mu) / jnp.sqrt(var + 1e-5) * g_ref[...] + b_ref[...]
    return jnp.maximum(xb, 0.0) + prev_ref[...]


def _tc_mid_body(x1p_ref, z_ref, prev_ref, g_ref, b_ref, wl_ref, bl_ref,
                 asd_ref, xn_ref, h_ref, ssd_ref):
    xn = _bn_relu_res(x1p_ref, z_ref, prev_ref, g_ref, b_ref)
    xn_ref[...] = xn
    h = _dot(xn, wl_ref[...]) + bl_ref[...]
    h_ref[...] = h
    ssd_ref[...] = _dot(h, asd_ref[...])


def _tc_final_body(x1p_ref, z_ref, prev_ref, g_ref, b_ref, wr1_ref, br1_ref,
                   wr2_ref, br2_ref, out_ref):
    xn = _bn_relu_res(x1p_ref, z_ref, prev_ref, g_ref, b_ref)
    t = _dot(xn, wr1_ref[...]) + br1_ref[...]
    sg = 1.0 / (1.0 + jnp.exp(-t))
    out_ref[...] = _dot(sg, wr2_ref[...]) + br2_ref[...]


def _vmesh():
    return plsc.VectorSubcoreMesh(core_axis_name="c", subcore_axis_name="s")


def _sc_params():
    cp = pltpu.CompilerParams()
    if "needs_layout_passes" in pltpu.CompilerParams.__dataclass_fields__:
        cp = dataclasses.replace(cp, needs_layout_passes=False)
    return cp


def _sc_embed_body(emb_hbm, attr2_hbm, e_hbm, attrb, rows, sem):
    wid = lax.axis_index("c") * _NS + lax.axis_index("s")
    nrows = _E // 128

    @pl.loop(wid, nrows, step=_NW)
    def _blk(r):
        pltpu.sync_copy(attr2_hbm.at[r], attrb)
        pltpu.async_copy(emb_hbm.at[attrb], rows, sem).wait()
        pltpu.sync_copy(rows, e_hbm.at[pl.ds(r * 128, 128)])


def _sc_att_body(ssd_hbm, te_hbm, src_hbm, dst_hbm, attr_hbm, w_hbm, z_hbm,
                 ssd_v, te_v, src_v, dst_v, attr_v, w_v, zp_v, zsh, acc_v,
                 tmp_v):
    cid = lax.axis_index("c")
    sid = lax.axis_index("s")
    base = (cid * _NS + sid) * _CH
    pltpu.sync_copy(ssd_hbm, ssd_v)
    pltpu.sync_copy(te_hbm, te_v)
    pltpu.sync_copy(src_hbm.at[pl.ds(base, _CH)], src_v)
    pltpu.sync_copy(dst_hbm.at[pl.ds(base, _CH)], dst_v)
    pltpu.sync_copy(attr_hbm.at[pl.ds(base, _CH)], attr_v)
    zf = jnp.zeros((16,), _F32)

    @pl.loop(0, _NPAD, step=16)
    def _zero(i):
        zp_v[pl.ds(i, 16)] = zf

    @pl.loop(0, _CH, step=16)
    def _edge(k):
        s16 = src_v[pl.ds(k, 16)]
        d16 = dst_v[pl.ds(k, 16)]
        a16 = attr_v[pl.ds(k, 16)]
        sv = plsc.load_gather(ssd_v, [s16 * 2])
        dv = plsc.load_gather(ssd_v, [d16 * 2 + 1])
        tv = plsc.load_gather(te_v, [a16])
        l = sv + dv + tv
        l = jnp.where(l > 0.0, l, 0.2 * l)
        w = jnp.exp(l)
        w_v[pl.ds(k, 16)] = w
        plsc.addupdate_scatter(zp_v, [d16], w)

    pltpu.sync_copy(w_v, w_hbm.at[pl.ds(base, _CH)])
    pltpu.sync_copy(zp_v, zsh.at[sid])
    plsc.subcore_barrier()
    off = sid * _SL
    pltpu.sync_copy(zsh.at[0, pl.ds(off, _SL)], acc_v)

    @pl.loop(1, _NS)
    def _row(r):
        pltpu.sync_copy(zsh.at[r, pl.ds(off, _SL)], tmp_v)

        @pl.loop(0, _SL, step=16)
        def _acc(j):
            acc_v[pl.ds(j, 16)] = acc_v[pl.ds(j, 16)] + tmp_v[pl.ds(j, 16)]

    pltpu.sync_copy(acc_v, z_hbm.at[cid, pl.ds(off, _SL)])


def _sc_msg_body(a_hbm, src_hbm, dst_hbm, e_hbm, h_hbm,
                 x1_hbm, sc1, al1, db0, db1,
                 hr0, er0, hr1, er1, x1_sh, sem0, sem1):
    cid = lax.axis_index("c")
    sid = lax.axis_index("s")
    wid = cid * _NS + sid
    base = wid * _CH
    off = sid * _SL
    nblk = _CH // _BO
    cblk = _CBK
    zf = jnp.zeros((16,), _F32)
    bufs = [(db0, hr0, er0, sem0), (db1, hr1, er1, sem1)]

    @pl.loop(0, _BO)
    def _zr(r):
        for c in range(8):
            hr0[r, pl.ds(16 * c, 16)] = zf
            er0[r, pl.ds(16 * c, 16)] = zf

    @pl.loop(0, _SL // _BO, step=2)
    def _zcp(t):
        pltpu.sync_copy(hr0, x1_sh.at[pl.ds(off + t * _BO, _BO)])
        pltpu.sync_copy(er0, x1_sh.at[pl.ds(off + (t + 1) * _BO, _BO)])

    plsc.subcore_barrier()

    def prefetch(buf, j, cb):
        db, hr, er, sem = buf
        pltpu.sync_copy(dst_hbm.at[pl.ds(cb + j * _BO, _BO)], db)
        pltpu.async_copy(h_hbm.at[sc1.at[pl.ds(j * _BO, _BO)]], hr, sem)
        pltpu.async_copy(e_hbm.at[pl.ds(cb + j * _BO, _BO)], er, sem)

    def consume(buf, j, cb):
        db, hr, er, sem = buf
        pltpu.make_async_copy(h_hbm.at[sc1.at[pl.ds(j * _BO, _BO)]],
                              hr, sem).wait()
        pltpu.make_async_copy(e_hbm.at[pl.ds(cb + j * _BO, _BO)],
                              er, sem).wait()

        @pl.loop(0, _BO)
        def _edge(k):
            asp = plsc.load_gather(al1, [jnp.zeros((16,), _I32) + j * _BO + k])
            for c in range(8):
                sl = pl.ds(16 * c, 16)
                hr[k, sl] = (hr[k, sl] + er[k, sl]) * asp

        pltpu.sync_copy(hr, x1_sh.at[db], add=True)

    for q in range(nblk // cblk):
        cb = base + q * cblk * _BO
        pltpu.sync_copy(src_hbm.at[pl.ds(cb, cblk * _BO)], sc1)
        pltpu.sync_copy(a_hbm.at[pl.ds(cb, cblk * _BO)], al1)
        prefetch(bufs[0], 0, cb)

        @pl.loop(0, cblk - 1, step=2)
        def _pair(g):
            for b in range(2):
                prefetch(bufs[1 - b], g + b + 1, cb)
                consume(bufs[b], g + b, cb)

        consume(bufs[0], cblk - 1, cb)

    plsc.subcore_barrier()
    pltpu.sync_copy(x1_sh.at[pl.ds(off, _SL)], x1_hbm.at[cid, pl.ds(off, _SL)])


def _sc_embed(emb, attr2):
    f = pl.kernel(
        _sc_embed_body,
        out_type=jax.ShapeDtypeStruct((_E, _D), _F32),
        mesh=_vmesh(),
        compiler_params=_sc_params(),
        scratch_types=[
            pltpu.VMEM((128,), _I32),
            pltpu.VMEM((128, _D), _F32),
            pltpu.SemaphoreType.DMA,
        ],
    )
    return f(emb, attr2)


def _sc_att(ssd, te_i, src, dst, attr):
    f = pl.kernel(
        _sc_att_body,
        out_type=(jax.ShapeDtypeStruct((_E,), _F32),
                  jax.ShapeDtypeStruct((_NC, _NPAD), _F32)),
        mesh=_vmesh(),
        compiler_params=_sc_params(),
        scratch_types=[
            pltpu.VMEM((2 * _N,), _F32),
            pltpu.VMEM((_VPAD,), _F32),
            pltpu.VMEM((_CH,), _I32),
            pltpu.VMEM((_CH,), _I32),
            pltpu.VMEM((_CH,), _I32),
            pltpu.VMEM((_CH,), _F32),
            pltpu.VMEM((_NPAD,), _F32),
            pltpu.VMEM_SHARED((_NS, _NPAD), _F32),
            pltpu.VMEM((_SL,), _F32),
            pltpu.VMEM((_SL,), _F32),
        ],
    )
    return f(ssd, te_i, src, dst, attr)


def _sc_msg(alpha, src, dst, e, h):
    f = pl.kernel(
        _sc_msg_body,
        out_type=jax.ShapeDtypeStruct((_NC, _NPAD, _D), _F32),
        mesh=_vmesh(),
        compiler_params=_sc_params(),
        scratch_types=[
            pltpu.VMEM((_CBK * _BO,), _I32),
            pltpu.VMEM((_CBK * _BO,), _F32),
            pltpu.VMEM((_BO,), _I32),
            pltpu.VMEM((_BO,), _I32),
            pltpu.VMEM((_BO, _D), _F32),
            pltpu.VMEM((_BO, _D), _F32),
            pltpu.VMEM((_BO, _D), _F32),
            pltpu.VMEM((_BO, _D), _F32),
            pltpu.VMEM_SHARED((_NPAD, _D), _F32),
            pltpu.SemaphoreType.DMA,
            pltpu.SemaphoreType.DMA,
        ],
    )
    return f(alpha, src, dst, e, h)


def kernel(x, edge_index, edge_attr, W_ne, b_ne, emb, Wl, bl, a_src, a_dst,
           a_e, gamma, beta, W_r1, b_r1, W_r2, b_r2):
    src = edge_index[0]
    dst = edge_index[1]
    attr = edge_attr
    attr2 = attr.reshape(_E // 128, 128)
    embp = jnp.pad(emb, ((0, _VPAD - emb.shape[0]), (0, 0)))
    asd = jnp.stack([a_src, a_dst], axis=2)
    r2 = lambda v: v.reshape(1, -1)

    xe, te, h, ssd = pl.pallas_call(
        _tc_encode_body,
        out_shape=[
            jax.ShapeDtypeStruct((_N, _D), _F32),
            jax.ShapeDtypeStruct((2, _VPAD), _F32),
            jax.ShapeDtypeStruct((_N, _D), _F32),
            jax.ShapeDtypeStruct((_N, 2), _F32),
        ],
    )(x, W_ne, r2(b_ne), embp, a_e, Wl[0], r2(bl[0]), asd[0])

    e = pl.pallas_call(
        _tc_gather_body,
        grid=(_E // _EB,),
        in_specs=[
            pl.BlockSpec((_EB,), lambda i: (i,)),
            pl.BlockSpec((_VPAD, _D), lambda i: (0, 0)),
        ],
        out_specs=pl.BlockSpec((_EB, _D), lambda i: (i, 0)),
        out_shape=jax.ShapeDtypeStruct((_E, _D), _F32),
    )(attr, embp)

    prev = xe
    out = None
    for i in range(2):
        w, z = _sc_att(ssd.reshape(2 * _N), te[i], src, dst, attr)
        x1p = _sc_msg(w, src, dst, e, h)
        if i == 0:
            prev, h, ssd = pl.pallas_call(
                _tc_mid_body,
                out_shape=[
                    jax.ShapeDtypeStruct((_N, _D), _F32),
                    jax.ShapeDtypeStruct((_N, _D), _F32),
                    jax.ShapeDtypeStruct((_N, 2), _F32),
                ],
            )(x1p, z, prev, r2(gamma[i]), r2(beta[i]), Wl[1], r2(bl[1]),
              asd[1])
        else:
            out = pl.pallas_call(
                _tc_final_body,
                out_shape=jax.ShapeDtypeStruct((_N, _D), _F32),
            )(x1p, z, prev, r2(gamma[i]), r2(beta[i]), W_r1, r2(b_r1), W_r2,
              r2(b_r2))
    return (out, e)


# speedup vs baseline: 1.2660x; 1.2660x over previous
import dataclasses

import jax
import jax.numpy as jnp
from jax import lax
from jax.experimental import pallas as pl
from jax.experimental.pallas import tpu as pltpu
from jax.experimental.pallas import tpu_sc as plsc

_N = 10000
_E = 320000
_D = 128
_NPAD = 10240
_NC = 2
_NS = 16
_NW = _NC * _NS
_CH = _E // _NW
_SL = _NPAD // _NS
_VPAD = 256
_EB = 512
_BO = 80
_CB = 2000
_CBK = 25
_F32 = jnp.float32
_I32 = jnp.int32
_HI = lax.Precision.HIGHEST


def _dot(a, b):
    return jnp.dot(a, b, preferred_element_type=_F32, precision=_HI)


def _tc_encode_body(x_ref, wne_ref, bne_ref, embp_ref, ae_ref, wl0_ref,
                    bl0_ref, asd0_ref, xe_ref, te_ref, h0_ref, ssd0_ref):
    xe = _dot(x_ref[...], wne_ref[...]) + bne_ref[...]
    xe_ref[...] = xe
    te_ref[...] = lax.dot_general(ae_ref[...], embp_ref[...],
                                  (((1,), (1,)), ((), ())),
                                  precision=_HI, preferred_element_type=_F32)
    h0 = _dot(xe, wl0_ref[...]) + bl0_ref[...]
    h0_ref[...] = h0
    ssd0_ref[...] = _dot(h0, asd0_ref[...])


def _tc_gather_body(attr_ref, embp_ref, e_ref):
    av = attr_ref[...]
    iota = lax.broadcasted_iota(_I32, (_EB, _VPAD), 1)
    oh = jnp.where(av[:, None] == iota, 1.0, 0.0).astype(_F32)
    e_ref[...] = _dot(oh, embp_ref[...])


def _bn_relu_res(x1p_ref, z_ref, prev_ref, g_ref, b_ref):
    zt = z_ref[0, :_N] + z_ref[1, :_N] + 1e-16
    s = (x1p_ref[0, :_N, :] + x1p_ref[1, :_N, :]) / zt[:, None]
    mu = jnp.mean(s, axis=0, keepdims=True)
    var = jnp.mean((s - mu) ** 2, axis=0, keepdims=True)
    xb = (s - mu) / jnp.sqrt(var + 1e-5) * g_ref[...] + b_ref[...]
    return jnp.maximum(xb, 0.0) + prev_ref[...]


def _tc_mid_body(x1p_ref, z_ref, prev_ref, g_ref, b_ref, wl_ref, bl_ref,
                 asd_ref, xn_ref, h_ref, ssd_ref):
    xn = _bn_relu_res(x1p_ref, z_ref, prev_ref, g_ref, b_ref)
    xn_ref[...] = xn
    h = _dot(xn, wl_ref[...]) + bl_ref[...]
    h_ref[...] = h
    ssd_ref[...] = _dot(h, asd_ref[...])


def _tc_final_body(x1p_ref, z_ref, prev_ref, g_ref, b_ref, wr1_ref, br1_ref,
                   wr2_ref, br2_ref, out_ref):
    xn = _bn_relu_res(x1p_ref, z_ref, prev_ref, g_ref, b_ref)
    t = _dot(xn, wr1_ref[...]) + br1_ref[...]
    sg = 1.0 / (1.0 + jnp.exp(-t))
    out_ref[...] = _dot(sg, wr2_ref[...]) + br2_ref[...]


def _vmesh():
    return plsc.VectorSubcoreMesh(core_axis_name="c", subcore_axis_name="s")


def _sc_params():
    cp = pltpu.CompilerParams()
    if "needs_layout_passes" in pltpu.CompilerParams.__dataclass_fields__:
        cp = dataclasses.replace(cp, needs_layout_passes=False)
    return cp


def _sc_embed_body(emb_hbm, attr2_hbm, e_hbm, attrb, rows, sem):
    wid = lax.axis_index("c") * _NS + lax.axis_index("s")
    nrows = _E // 128

    @pl.loop(wid, nrows, step=_NW)
    def _blk(r):
        pltpu.sync_copy(attr2_hbm.at[r], attrb)
        pltpu.async_copy(emb_hbm.at[attrb], rows, sem).wait()
        pltpu.sync_copy(rows, e_hbm.at[pl.ds(r * 128, 128)])


def _sc_att_body(ssd_hbm, te_hbm, src_hbm, dst_hbm, attr_hbm, w_hbm, z_hbm,
                 ssd_v, te_v, src_v, dst_v, attr_v, w_v, zp_v, zsh, acc_v,
                 tmp_v):
    cid = lax.axis_index("c")
    sid = lax.axis_index("s")
    base = (cid * _NS + sid) * _CH
    pltpu.sync_copy(ssd_hbm, ssd_v)
    pltpu.sync_copy(te_hbm, te_v)
    pltpu.sync_copy(src_hbm.at[pl.ds(base, _CH)], src_v)
    pltpu.sync_copy(dst_hbm.at[pl.ds(base, _CH)], dst_v)
    pltpu.sync_copy(attr_hbm.at[pl.ds(base, _CH)], attr_v)
    zf = jnp.zeros((16,), _F32)

    @pl.loop(0, _NPAD, step=16)
    def _zero(i):
        zp_v[pl.ds(i, 16)] = zf

    @pl.loop(0, _CH, step=16)
    def _edge(k):
        s16 = src_v[pl.ds(k, 16)]
        d16 = dst_v[pl.ds(k, 16)]
        a16 = attr_v[pl.ds(k, 16)]
        sv = plsc.load_gather(ssd_v, [s16 * 2])
        dv = plsc.load_gather(ssd_v, [d16 * 2 + 1])
        tv = plsc.load_gather(te_v, [a16])
        l = sv + dv + tv
        l = jnp.where(l > 0.0, l, 0.2 * l)
        w = jnp.exp(l)
        w_v[pl.ds(k, 16)] = w
        plsc.addupdate_scatter(zp_v, [d16], w)

    pltpu.sync_copy(w_v, w_hbm.at[pl.ds(base, _CH)])
    pltpu.sync_copy(zp_v, zsh.at[sid])
    plsc.subcore_barrier()
    off = sid * _SL
    pltpu.sync_copy(zsh.at[0, pl.ds(off, _SL)], acc_v)

    @pl.loop(1, _NS)
    def _row(r):
        pltpu.sync_copy(zsh.at[r, pl.ds(off, _SL)], tmp_v)

        @pl.loop(0, _SL, step=16)
        def _acc(j):
            acc_v[pl.ds(j, 16)] = acc_v[pl.ds(j, 16)] + tmp_v[pl.ds(j, 16)]

    pltpu.sync_copy(acc_v, z_hbm.at[cid, pl.ds(off, _SL)])


def _sc_msg_body(a_hbm, src_hbm, dst_hbm, attr_hbm, h_hbm, emb_hbm,
                 x1_hbm, sc1, at1, al1, db0, db1,
                 hr0, er0, hr1, er1, x1_sh, sem0, sem1):
    cid = lax.axis_index("c")
    sid = lax.axis_index("s")
    wid = cid * _NS + sid
    base = wid * _CH
    off = sid * _SL
    nblk = _CH // _BO
    cblk = _CBK
    zf = jnp.zeros((16,), _F32)
    bufs = [(db0, hr0, er0, sem0), (db1, hr1, er1, sem1)]

    @pl.loop(0, _BO)
    def _zr(r):
        for c in range(8):
            hr0[r, pl.ds(16 * c, 16)] = zf
            er0[r, pl.ds(16 * c, 16)] = zf

    @pl.loop(0, _SL // _BO, step=2)
    def _zcp(t):
        pltpu.sync_copy(hr0, x1_sh.at[pl.ds(off + t * _BO, _BO)])
        pltpu.sync_copy(er0, x1_sh.at[pl.ds(off + (t + 1) * _BO, _BO)])

    plsc.subcore_barrier()

    def prefetch(buf, j, cb):
        db, hr, er, sem = buf
        pltpu.sync_copy(dst_hbm.at[pl.ds(cb + j * _BO, _BO)], db)
        pltpu.async_copy(h_hbm.at[sc1.at[pl.ds(j * _BO, _BO)]], hr, sem)
        pltpu.async_copy(emb_hbm.at[at1.at[pl.ds(j * _BO, _BO)]], er, sem)

    def consume(buf, j):
        db, hr, er, sem = buf
        pltpu.make_async_copy(h_hbm.at[sc1.at[pl.ds(j * _BO, _BO)]],
                              hr, sem).wait()
        pltpu.make_async_copy(emb_hbm.at[at1.at[pl.ds(j * _BO, _BO)]],
                              er, sem).wait()

        @pl.loop(0, _BO)
        def _edge(k):
            asp = plsc.load_gather(al1, [jnp.zeros((16,), _I32) + j * _BO + k])
            for c in range(8):
                sl = pl.ds(16 * c, 16)
                hr[k, sl] = (hr[k, sl] + er[k, sl]) * asp

        pltpu.sync_copy(hr, x1_sh.at[db], add=True)

    for q in range(nblk // cblk):
        cb = base + q * cblk * _BO
        pltpu.sync_copy(src_hbm.at[pl.ds(cb, cblk * _BO)], sc1)
        pltpu.sync_copy(attr_hbm.at[pl.ds(cb, cblk * _BO)], at1)
        pltpu.sync_copy(a_hbm.at[pl.ds(cb, cblk * _BO)], al1)
        prefetch(bufs[0], 0, cb)

        @pl.loop(0, cblk - 1, step=2)
        def _pair(g):
            for b in range(2):
                prefetch(bufs[1 - b], g + b + 1, cb)
                consume(bufs[b], g + b)

        consume(bufs[0], cblk - 1)

    plsc.subcore_barrier()
    pltpu.sync_copy(x1_sh.at[pl.ds(off, _SL)], x1_hbm.at[cid, pl.ds(off, _SL)])


def _sc_embed(emb, attr2):
    f = pl.kernel(
        _sc_embed_body,
        out_type=jax.ShapeDtypeStruct((_E, _D), _F32),
        mesh=_vmesh(),
        compiler_params=_sc_params(),
        scratch_types=[
            pltpu.VMEM((128,), _I32),
            pltpu.VMEM((128, _D), _F32),
            pltpu.SemaphoreType.DMA,
        ],
    )
    return f(emb, attr2)


def _sc_att(ssd, te_i, src, dst, attr):
    f = pl.kernel(
        _sc_att_body,
        out_type=(jax.ShapeDtypeStruct((_E,), _F32),
                  jax.ShapeDtypeStruct((_NC, _NPAD), _F32)),
        mesh=_vmesh(),
        compiler_params=_sc_params(),
        scratch_types=[
            pltpu.VMEM((2 * _N,), _F32),
            pltpu.VMEM((_VPAD,), _F32),
            pltpu.VMEM((_CH,), _I32),
            pltpu.VMEM((_CH,), _I32),
            pltpu.VMEM((_CH,), _I32),
            pltpu.VMEM((_CH,), _F32),
            pltpu.VMEM((_NPAD,), _F32),
            pltpu.VMEM_SHARED((_NS, _NPAD), _F32),
            pltpu.VMEM((_SL,), _F32),
            pltpu.VMEM((_SL,), _F32),
        ],
    )
    return f(ssd, te_i, src, dst, attr)


def _sc_msg(alpha, src, dst, attr, h, emb):
    f = pl.kernel(
        _sc_msg_body,
        out_type=jax.ShapeDtypeStruct((_NC, _NPAD, _D), _F32),
        mesh=_vmesh(),
        compiler_params=_sc_params(),
        scratch_types=[
            pltpu.VMEM((_CBK * _BO,), _I32),
            pltpu.VMEM((_CBK * _BO,), _I32),
            pltpu.VMEM((_CBK * _BO,), _F32),
            pltpu.VMEM((_BO,), _I32),
            pltpu.VMEM((_BO,), _I32),
            pltpu.VMEM((_BO, _D), _F32),
            pltpu.VMEM((_BO, _D), _F32),
            pltpu.VMEM((_BO, _D), _F32),
            pltpu.VMEM((_BO, _D), _F32),
            pltpu.VMEM_SHARED((_NPAD, _D), _F32),
            pltpu.SemaphoreType.DMA,
            pltpu.SemaphoreType.DMA,
        ],
    )
    return f(alpha, src, dst, attr, h, emb)


def kernel(x, edge_index, edge_attr, W_ne, b_ne, emb, Wl, bl, a_src, a_dst,
           a_e, gamma, beta, W_r1, b_r1, W_r2, b_r2):
    src = edge_index[0]
    dst = edge_index[1]
    attr = edge_attr
    attr2 = attr.reshape(_E // 128, 128)
    embp = jnp.pad(emb, ((0, _VPAD - emb.shape[0]), (0, 0)))
    asd = jnp.stack([a_src, a_dst], axis=2)
    r2 = lambda v: v.reshape(1, -1)

    xe, te, h, ssd = pl.pallas_call(
        _tc_encode_body,
        out_shape=[
            jax.ShapeDtypeStruct((_N, _D), _F32),
            jax.ShapeDtypeStruct((2, _VPAD), _F32),
            jax.ShapeDtypeStruct((_N, _D), _F32),
            jax.ShapeDtypeStruct((_N, 2), _F32),
        ],
    )(x, W_ne, r2(b_ne), embp, a_e, Wl[0], r2(bl[0]), asd[0])

    e = pl.pallas_call(
        _tc_gather_body,
        grid=(_E // _EB,),
        in_specs=[
            pl.BlockSpec((_EB,), lambda i: (i,)),
            pl.BlockSpec((_VPAD, _D), lambda i: (0, 0)),
        ],
        out_specs=pl.BlockSpec((_EB, _D), lambda i: (i, 0)),
        out_shape=jax.ShapeDtypeStruct((_E, _D), _F32),
    )(attr, embp)

    prev = xe
    out = None
    for i in range(2):
        w, z = _sc_att(ssd.reshape(2 * _N), te[i], src, dst, attr)
        x1p = _sc_msg(w, src, dst, attr, h, emb)
        if i == 0:
            prev, h, ssd = pl.pallas_call(
                _tc_mid_body,
                out_shape=[
                    jax.ShapeDtypeStruct((_N, _D), _F32),
                    jax.ShapeDtypeStruct((_N, _D), _F32),
                    jax.ShapeDtypeStruct((_N, 2), _F32),
                ],
            )(x1p, z, prev, r2(gamma[i]), r2(beta[i]), Wl[1], r2(bl[1]),
              asd[1])
        else:
            out = pl.pallas_call(
                _tc_final_body,
                out_shape=jax.ShapeDtypeStruct((_N, _D), _F32),
            )(x1p, z, prev, r2(gamma[i]), r2(beta[i]), W_r1, r2(b_r1), W_r2,
              r2(b_r2))
    return (out, e)
